# baseline (speedup 1.0000x reference)
import jax
import jax.numpy as jnp
from jax import lax
from jax.experimental import pallas as pl
from jax.experimental.pallas import tpu as pltpu
from jax.experimental.pallas import tpu_sc as plsc

DIM = 1024
T = 20
LANES = 16
CHUNKS = DIM // LANES
NEWTON_ITERS = 3
EPS = 1e-5
NSLOT = 4
CHUNK_BAGS = 104

_GATHER_DN = lax.GatherDimensionNumbers(
    offset_dims=(), collapsed_slice_dims=(0,), start_index_map=(0,))


def _lane_shuffle(v, idx):
    return lax.gather(v, idx[:, None], _GATHER_DN, slice_sizes=(1,),
                      mode=lax.GatherScatterMode.PROMISE_IN_BOUNDS)


def _lane_allreduce_sum(v):
    lanes = lax.iota(jnp.int32, LANES)
    for shift in (1, 2, 4, 8):
        v = v + _lane_shuffle(v, lanes ^ shift)
    return v


def _rsqrt16(a):
    xi = lax.bitcast_convert_type(a, jnp.int32)
    yi = jnp.int32(0x5F3759DF) - (xi >> 1)
    y = lax.bitcast_convert_type(yi, jnp.float32)
    half = a * 0.5
    for _ in range(NEWTON_ITERS):
        y = y * (1.5 - half * y * y)
    return y


def _make_sc_kernel(num_bags, bags_per_w):
    mesh = plsc.VectorSubcoreMesh(core_axis_name="c", subcore_axis_name="s")
    nc = mesh.num_cores
    n_chunks = bags_per_w // CHUNK_BAGS

    def run(x, table):
        @pl.kernel(
            out_type=jax.ShapeDtypeStruct((num_bags, DIM), jnp.float32),
            mesh=mesh,
            scratch_types=[
                pltpu.VMEM((2, CHUNK_BAGS, T), jnp.int32),
                pltpu.VMEM((NSLOT, T, DIM), jnp.float32),
                pltpu.VMEM((NSLOT * DIM,), jnp.float32),
            ] + [pltpu.SemaphoreType.DMA] * (2 * NSLOT + 2),
            compiler_params=pltpu.CompilerParams(use_tc_tiling_on_sc=True),
        )
        def body(idx_hbm, table_hbm, out_hbm, idx_v, rows_v, row_v, *sems):
            gsems = sems[:NSLOT]
            osems = sems[NSLOT:2 * NSLOT]
            isems = sems[2 * NSLOT:]
            wid = lax.axis_index("s") * nc + lax.axis_index("c")
            base = wid * bags_per_w
            zeros = jnp.zeros((LANES,), jnp.float32)

            def idx_ref_local(ib, jj):
                return idx_v.at[ib, jj]

            def start_gather(ib, jj, slot):
                pltpu.async_copy(
                    table_hbm.at[idx_ref_local(ib, jj)], rows_v.at[slot],
                    gsems[slot])

            def do_bag(jj, g, slot, ib, cross_guard):
                pltpu.make_async_copy(
                    table_hbm.at[idx_ref_local(ib, jj)], rows_v.at[slot],
                    gsems[slot]).wait()

                @pl.when(g >= base + NSLOT)
                def _():
                    pltpu.make_async_copy(
                        row_v.at[pl.ds(slot * DIM, DIM)],
                        out_hbm.at[g - NSLOT], osems[slot]).wait()

                pslot = (slot + NSLOT - 1) % NSLOT
                jp = jj + NSLOT - 1

                @pl.when(jp < CHUNK_BAGS)
                def _():
                    start_gather(ib, jp, pslot)

                if cross_guard is not None:
                    @pl.when(jnp.logical_and(jp >= CHUNK_BAGS, cross_guard))
                    def _():
                        start_gather(1 - ib, jp - CHUNK_BAGS, pslot)

                def chunk_body(cc, carry):
                    vsum, vsq = carry
                    vals = [rows_v[slot, t, pl.ds(cc * LANES, LANES)]
                            for t in range(T)]
                    while len(vals) > 1:
                        nxt = [vals[k] + vals[k + 1]
                               for k in range(0, len(vals) - 1, 2)]
                        if len(vals) % 2:
                            nxt[-1] = nxt[-1] + vals[-1]
                        vals = nxt
                    s = vals[0]
                    row_v[pl.ds(slot * DIM + cc * LANES, LANES)] = s
                    return (vsum + s, vsq + s * s)

                vsum, vsq = lax.fori_loop(
                    0, CHUNKS, chunk_body, (zeros, zeros), unroll=2)
                mean = _lane_allreduce_sum(vsum) * (1.0 / DIM)
                ex2 = _lane_allreduce_sum(vsq) * (1.0 / DIM)
                rstd = _rsqrt16(ex2 - mean * mean + EPS)

                def norm_body(cc, _):
                    o = pl.ds(slot * DIM + cc * LANES, LANES)
                    row_v[o] = (row_v[o] - mean) * rstd
                    return 0

                lax.fori_loop(0, CHUNKS, norm_body, 0)
                pltpu.async_copy(
                    row_v.at[pl.ds(slot * DIM, DIM)], out_hbm.at[g],
                    osems[slot])

            def run_chunk(cp, ib, cross_guard):
                c = cp * 2 + ib
                @pl.when(c + 1 < n_chunks)
                def _():
                    pltpu.make_async_copy(
                        idx_hbm.at[pl.ds(base + (c + 1) * CHUNK_BAGS,
                                         CHUNK_BAGS)],
                        idx_v.at[1 - ib], isems[1 - ib]).wait()

                cbase = base + c * CHUNK_BAGS

                @pl.loop(0, CHUNK_BAGS, step=NSLOT)
                def _(jj0):
                    for b in range(NSLOT):
                        do_bag(jj0 + b, cbase + jj0 + b, b, ib, cross_guard)

                @pl.when(c + 2 < n_chunks)
                def _():
                    pltpu.async_copy(
                        idx_hbm.at[pl.ds(base + (c + 2) * CHUNK_BAGS,
                                         CHUNK_BAGS)],
                        idx_v.at[ib], isems[ib])

            pltpu.sync_copy(
                idx_hbm.at[pl.ds(base, CHUNK_BAGS)], idx_v.at[0])
            pltpu.async_copy(
                idx_hbm.at[pl.ds(base + CHUNK_BAGS, CHUNK_BAGS)],
                idx_v.at[1], isems[1])
            for s in range(NSLOT - 1):
                start_gather(0, s, s)

            @pl.loop(0, n_chunks // 2)
            def _(cp):
                run_chunk(cp, 0, cross_guard=jnp.bool_(True))
                run_chunk(cp, 1, cross_guard=cp < n_chunks // 2 - 1)

            for s in range(NSLOT):
                g = base + bags_per_w - NSLOT + s
                pltpu.make_async_copy(
                    row_v.at[pl.ds(s * DIM, DIM)], out_hbm.at[g],
                    osems[s]).wait()

        return body(x, table)

    return run


_NUM_WORKERS = 32
_sc_run = None


def kernel(x, table):
    global _sc_run
    B, F_, t = x.shape
    num_bags = B * F_
    if _sc_run is None:
        _sc_run = _make_sc_kernel(num_bags, num_bags // _NUM_WORKERS)
    out = _sc_run(x.reshape(num_bags, t), table)
    return out.reshape(B, F_, table.shape[1])
